# baseline (speedup 1.0000x reference)
import jax
import jax.numpy as jnp
from jax import lax
from jax.experimental import pallas as pl
from jax.experimental.pallas import tpu as pltpu
from jax.experimental.pallas import tpu_sc as plsc

N = 10000
E = 320000
FEAT = 128
HID = 128
HHID = HID // 2

NC = 2
NS = 16
E_T = E // NS
CHUNK = 80
BLK_CH = 25
NBLK = E_T // (BLK_CH * CHUNK)
N_PAD = 10240
ROWS_T = N_PAD // NS
TAB_STRIDE = 624

_mesh = plsc.VectorSubcoreMesh(core_axis_name="c", subcore_axis_name="s")
_sc_params = pltpu.CompilerParams(needs_layout_passes=False,
                                  use_tc_tiling_on_sc=False)


E_T32 = E // (NC * NS)
E_T16 = E // NS
NROW = 640
RPT = NROW // NS


def _frsqrt(x):
    i = lax.bitcast_convert_type(x, jnp.int32)
    i = jnp.int32(0x5F3759DF) - (i >> 1)
    y = lax.bitcast_convert_type(i, jnp.float32)
    for _ in range(3):
        y = y * (1.5 - 0.5 * x * y * y)
    return y


def _coef_body(src_hbm, dst_hbm, ew_hbm, coef_hbm, dis2_hbm,
               dst_v, ew_v, src_v, hist, dis_l, dstr, ystr, d2str,
               cf_v, idx_v, zbuf, dacc, sdis, sem):
    c = lax.axis_index("c")
    s = lax.axis_index("s")

    pltpu.sync_copy(dst_hbm.at[s], dst_v)
    pltpu.sync_copy(ew_hbm.at[s], ew_v)
    pltpu.sync_copy(src_hbm.at[s, c], src_v)

    zero16 = jnp.zeros((16,), jnp.float32)

    def zhist(r, _):
        hist[r] = zero16
        return 0

    lax.fori_loop(0, NROW, zhist, 0)

    def zrow(r, _):
        zbuf[r] = zero16
        return 0

    lax.fori_loop(0, RPT, zrow, 0)

    def fidx(g, _):
        idx_v[pl.ds(g * 16, 16)] = lax.iota(jnp.int32, 16) + g * 16
        return 0

    lax.fori_loop(0, NROW // 16, fidx, 0)

    base = s * RPT
    pltpu.sync_copy(zbuf, dacc.at[pl.ds(base, RPT)])
    plsc.subcore_barrier()

    def accum(h):
        def grp(i, _):
            d16 = dst_v[h, pl.ds(i * 16, 16)]
            v16 = ew_v[h, pl.ds(i * 16, 16)]
            plsc.addupdate_scatter(hist, [d16 >> 4, d16 & 15], v16)
            return 0

        lax.fori_loop(0, E_T32 // 16, grp, 0)

    accum(0)
    accum(1)

    for k in range(5):
        pltpu.async_copy(hist.at[pl.ds(k * 128, 128)],
                         dacc.at[idx_v.at[pl.ds(k * 128, 128)]], sem,
                         add=True)
    for k in range(5):
        pltpu.make_async_copy(hist.at[pl.ds(0, 128)],
                              dacc.at[idx_v.at[pl.ds(0, 128)]], sem).wait()
    plsc.subcore_barrier()

    pltpu.sync_copy(dacc.at[pl.ds(base, RPT)], dstr)

    def dgrp(r, _):
        deg = dstr[r] + 1.0
        y = _frsqrt(deg)
        ystr[r] = y
        d2str[r] = y * y
        return 0

    lax.fori_loop(0, RPT, dgrp, 0)
    pltpu.sync_copy(ystr, sdis.at[pl.ds(base, RPT)])

    @pl.when(c == 0)
    def _():
        pltpu.sync_copy(d2str, dis2_hbm.at[pl.ds(base, RPT)])

    plsc.subcore_barrier()

    pltpu.sync_copy(sdis, dis_l)

    def cgrp(i, _):
        sl = pl.ds(i * 16, 16)
        s16 = src_v[sl]
        d16 = dst_v[c, sl]
        a = plsc.load_gather(dis_l, [s16 >> 4, s16 & 15])
        b = plsc.load_gather(dis_l, [d16 >> 4, d16 & 15])
        cf_v[sl] = ew_v[c, sl] * a * b
        return 0

    lax.fori_loop(0, E_T32 // 16, cgrp, 0)
    pltpu.sync_copy(cf_v, coef_hbm.at[s, c])


_coef_call = pl.kernel(
    _coef_body,
    out_type=(jax.ShapeDtypeStruct((NS, NC, E_T32), jnp.float32),
              jax.ShapeDtypeStruct((NROW, 16), jnp.float32)),
    mesh=_mesh,
    scratch_types=[
        pltpu.VMEM((NC, E_T32), jnp.int32),
        pltpu.VMEM((NC, E_T32), jnp.float32),
        pltpu.VMEM((E_T32,), jnp.int32),
        pltpu.VMEM((NROW, 16), jnp.float32),
        pltpu.VMEM((NROW, 16), jnp.float32),
        pltpu.VMEM((RPT, 16), jnp.float32),
        pltpu.VMEM((RPT, 16), jnp.float32),
        pltpu.VMEM((RPT, 16), jnp.float32),
        pltpu.VMEM((E_T32,), jnp.float32),
        pltpu.VMEM((NROW,), jnp.int32),
        pltpu.VMEM((RPT, 16), jnp.float32),
        pltpu.VMEM_SHARED((NROW, 16), jnp.float32),
        pltpu.VMEM_SHARED((NROW, 16), jnp.float32),
        pltpu.SemaphoreType.DMA,
    ],
    compiler_params=_sc_params,
)


def _spmm_body(g_hbm, src_hbm, dst_hbm, ew_hbm, out_hbm,
               src_b, dst_b, ew_b, g0, g1, s0, s1, tab, acc,
               sg0, sg1, ss0, ss1, si0, si1):
    _spmm_core(False, g_hbm, src_hbm, dst_hbm, ew_hbm, None, None, out_hbm,
               src_b, dst_b, ew_b, g0, g1, s0, s1, None, None, tab, acc,
               sg0, sg1, ss0, ss1, si0, si1)


def _spmm_comb_body(g_hbm, src_hbm, dst_hbm, ew_hbm, dis2_hbm, b_hbm, out_hbm,
                    src_b, dst_b, ew_b, g0, g1, s0, s1, d2l, b2l, tab, acc,
                    sg0, sg1, ss0, ss1, si0, si1):
    _spmm_core(True, g_hbm, src_hbm, dst_hbm, ew_hbm, dis2_hbm, b_hbm,
               out_hbm, src_b, dst_b, ew_b, g0, g1, s0, s1, d2l, b2l, tab,
               acc, sg0, sg1, ss0, ss1, si0, si1)


def _spmm_core(fuse_comb, g_hbm, src_hbm, dst_hbm, ew_hbm, dis2_hbm, b_hbm,
               out_hbm, src_b, dst_b, ew_b, g0, g1, s0, s1, d2l, b2l, tab,
               acc, sg0, sg1, ss0, ss1, si0, si1):
    c = lax.axis_index("c")
    s = lax.axis_index("s")
    sis = (si0, si1)

    def issue_idx(b, p):
        pltpu.async_copy(src_hbm.at[s, b], src_b.at[p], sis[p])
        pltpu.async_copy(dst_hbm.at[s, b], dst_b.at[p], sis[p])
        pltpu.async_copy(ew_hbm.at[s, b], ew_b.at[p], sis[p])

    def wait_idx(b, p):
        pltpu.make_async_copy(src_hbm.at[s, b], src_b.at[p], sis[p]).wait()
        pltpu.make_async_copy(dst_hbm.at[s, b], dst_b.at[p], sis[p]).wait()
        pltpu.make_async_copy(ew_hbm.at[s, b], ew_b.at[p], sis[p]).wait()

    issue_idx(0, 0)
    if fuse_comb:
        pltpu.sync_copy(dis2_hbm.at[pl.ds(RPT * s, RPT)], d2l)
        pltpu.sync_copy(b_hbm.at[c], b2l)

    tstart = s * TAB_STRIDE
    pltpu.sync_copy(g_hbm.at[c, pl.ds(tstart, 640)], tab.at[pl.ds(tstart, 640)])

    def zrow(i, _):
        for f in range(HHID // 16):
            s0[i, pl.ds(f * 16, 16)] = jnp.zeros((16,), jnp.float32)
        return 0

    lax.fori_loop(0, CHUNK, zrow, 0)
    base = s * ROWS_T
    for k in range(ROWS_T // CHUNK):
        pltpu.sync_copy(s0, acc.at[pl.ds(base + k * CHUNK, CHUNK)])
    plsc.subcore_barrier()

    def gather(p, j, buf, sem):
        pltpu.async_copy(tab.at[src_b.at[p, j]], buf, sem)

    def wait_gather(p, buf, sem):
        pltpu.make_async_copy(tab.at[src_b.at[p, 0]], buf, sem).wait()

    def scatter(p, j, buf, sem):
        pltpu.async_copy(buf, acc.at[dst_b.at[p, j]], sem, add=True)

    def wait_scatter(p, buf, sem):
        pltpu.make_async_copy(buf, acc.at[dst_b.at[p, 0]], sem).wait()

    def scale(p, j, gbuf, sbuf):
        def grp(g, _):
            ew16 = ew_b[p, j, pl.ds(g * 16, 16)]
            for l in range(16):
                wgt = jnp.full((16,), ew16[l])
                e = g * 16 + l
                for f in range(HHID // 16):
                    sl = pl.ds(f * 16, 16)
                    sbuf[e, sl] = gbuf[e, sl] * wgt
            return 0

        lax.fori_loop(0, CHUNK // 16, grp, 0)

    def process_block(p):
        gather(p, 0, g0, sg0)
        gather(p, 1, g1, sg1)

        wait_gather(p, g0, sg0)
        scale(p, 0, g0, s0)
        scatter(p, 0, s0, ss0)
        gather(p, 2, g0, sg0)
        wait_gather(p, g1, sg1)
        scale(p, 1, g1, s1)
        scatter(p, 1, s1, ss1)
        gather(p, 3, g1, sg1)

        def steady(i, _):
            j0 = 2 * i
            wait_gather(p, g0, sg0)
            wait_scatter(p, s0, ss0)
            scale(p, j0, g0, s0)
            scatter(p, j0, s0, ss0)
            gather(p, j0 + 2, g0, sg0)
            j1 = j0 + 1
            wait_gather(p, g1, sg1)
            wait_scatter(p, s1, ss1)
            scale(p, j1, g1, s1)
            scatter(p, j1, s1, ss1)
            gather(p, j1 + 2, g1, sg1)
            return 0

        lax.fori_loop(1, (BLK_CH - 3) // 2, steady, 0)

        wait_gather(p, g0, sg0)
        wait_scatter(p, s0, ss0)
        scale(p, BLK_CH - 3, g0, s0)
        scatter(p, BLK_CH - 3, s0, ss0)
        gather(p, BLK_CH - 1, g0, sg0)
        wait_gather(p, g1, sg1)
        wait_scatter(p, s1, ss1)
        scale(p, BLK_CH - 2, g1, s1)
        scatter(p, BLK_CH - 2, s1, ss1)
        wait_gather(p, g0, sg0)
        wait_scatter(p, s0, ss0)
        scale(p, BLK_CH - 1, g0, s0)
        scatter(p, BLK_CH - 1, s0, ss0)
        wait_scatter(p, s0, ss0)
        wait_scatter(p, s1, ss1)

    def pair_body(bb, _):
        b0 = 2 * bb
        b1 = b0 + 1
        bn = lax.rem(b0 + 2, NBLK)
        wait_idx(b0, 0)
        issue_idx(b1, 1)
        process_block(0)
        wait_idx(b1, 1)
        issue_idx(bn, 0)
        process_block(1)
        return 0

    lax.fori_loop(0, NBLK // 2, pair_body, 0)
    wait_idx(0, 0)

    plsc.subcore_barrier()
    if not fuse_comb:
        pltpu.sync_copy(acc.at[pl.ds(base, ROWS_T)],
                        out_hbm.at[c, pl.ds(base, ROWS_T)])
    else:
        for k in range(ROWS_T // CHUNK):
            rs = base + k * CHUNK
            pltpu.sync_copy(acc.at[pl.ds(rs, CHUNK)], g0)
            pltpu.sync_copy(tab.at[pl.ds(rs, CHUNK)], g1)

            def erow(g, _):
                d2v = d2l[5 * k + g]
                for l in range(16):
                    w = jnp.full((16,), d2v[l])
                    e = g * 16 + l
                    for f in range(HHID // 16):
                        sl = pl.ds(f * 16, 16)
                        s0[e, sl] = g0[e, sl] + w * g1[e, sl] + b2l[sl]
                return 0

            lax.fori_loop(0, CHUNK // 16, erow, 0)
            pltpu.sync_copy(s0, out_hbm.at[c, pl.ds(rs, CHUNK)])


_spmm_call = pl.kernel(
    _spmm_body,
    out_type=jax.ShapeDtypeStruct((NC, N_PAD, HHID), jnp.float32),
    mesh=_mesh,
    scratch_types=[
        pltpu.VMEM((2, BLK_CH, CHUNK), jnp.int32),
        pltpu.VMEM((2, BLK_CH, CHUNK), jnp.int32),
        pltpu.VMEM((2, BLK_CH, CHUNK), jnp.float32),
        pltpu.VMEM((CHUNK, HHID), jnp.float32),
        pltpu.VMEM((CHUNK, HHID), jnp.float32),
        pltpu.VMEM((CHUNK, HHID), jnp.float32),
        pltpu.VMEM((CHUNK, HHID), jnp.float32),
        pltpu.VMEM_SHARED((N_PAD, HHID), jnp.float32),
        pltpu.VMEM_SHARED((N_PAD, HHID), jnp.float32),
        pltpu.SemaphoreType.DMA,
        pltpu.SemaphoreType.DMA,
        pltpu.SemaphoreType.DMA,
        pltpu.SemaphoreType.DMA,
        pltpu.SemaphoreType.DMA,
        pltpu.SemaphoreType.DMA,
    ],
    compiler_params=_sc_params,
)


_spmm_comb_call = pl.kernel(
    _spmm_comb_body,
    out_type=jax.ShapeDtypeStruct((NC, N_PAD, HHID), jnp.float32),
    mesh=_mesh,
    scratch_types=[
        pltpu.VMEM((2, BLK_CH, CHUNK), jnp.int32),
        pltpu.VMEM((2, BLK_CH, CHUNK), jnp.int32),
        pltpu.VMEM((2, BLK_CH, CHUNK), jnp.float32),
        pltpu.VMEM((CHUNK, HHID), jnp.float32),
        pltpu.VMEM((CHUNK, HHID), jnp.float32),
        pltpu.VMEM((CHUNK, HHID), jnp.float32),
        pltpu.VMEM((CHUNK, HHID), jnp.float32),
        pltpu.VMEM((RPT, 16), jnp.float32),
        pltpu.VMEM((HHID,), jnp.float32),
        pltpu.VMEM_SHARED((N_PAD, HHID), jnp.float32),
        pltpu.VMEM_SHARED((N_PAD, HHID), jnp.float32),
        pltpu.SemaphoreType.DMA,
        pltpu.SemaphoreType.DMA,
        pltpu.SemaphoreType.DMA,
        pltpu.SemaphoreType.DMA,
        pltpu.SemaphoreType.DMA,
        pltpu.SemaphoreType.DMA,
    ],
    compiler_params=_sc_params,
)


_BR = 1000
_GRID = N // _BR


def _halves(v):
    return v[:, :HHID], v[:, HHID:]


def _mm_body(x_ref, w_ref, h_ref):
    h = jnp.dot(x_ref[...], w_ref[...], preferred_element_type=jnp.float32,
                precision=lax.Precision.HIGHEST)
    ha, hb = _halves(h)
    h_ref[0] = ha
    h_ref[1] = hb


def _mm_call(x, W1):
    return pl.pallas_call(
        _mm_body,
        grid=(_GRID,),
        in_specs=[
            pl.BlockSpec((_BR, FEAT), lambda i: (i, 0)),
            pl.BlockSpec((FEAT, HID), lambda i: (0, 0)),
        ],
        out_specs=pl.BlockSpec((2, _BR, HHID), lambda i: (0, i, 0)),
        out_shape=jax.ShapeDtypeStruct((2, N, HHID), jnp.float32),
    )(x, W1)


def _comb1_body(sa_ref, sb_ref, ha_ref, hb_ref, d2_ref, b_ref, w_ref, h2_ref):
    d2 = d2_ref[...]
    b = b_ref[...]
    ta = sa_ref[0] + ha_ref[0] * d2 + b[:, :HHID]
    tb = sb_ref[0] + hb_ref[0] * d2 + b[:, HHID:]
    o = jnp.maximum(jnp.concatenate([ta, tb], axis=1), 0.0)
    h2 = jnp.dot(o, w_ref[...], preferred_element_type=jnp.float32,
                 precision=lax.Precision.HIGHEST)
    h2a, h2b = _halves(h2)
    h2_ref[0] = h2a
    h2_ref[1] = h2b


def _comb1_call(s1, h1, dis2, b1, W2):
    return pl.pallas_call(
        _comb1_body,
        grid=(_GRID,),
        in_specs=[
            pl.BlockSpec((1, _BR, HHID), lambda i: (0, i, 0)),
            pl.BlockSpec((1, _BR, HHID), lambda i: (1, i, 0)),
            pl.BlockSpec((1, _BR, HHID), lambda i: (0, i, 0)),
            pl.BlockSpec((1, _BR, HHID), lambda i: (1, i, 0)),
            pl.BlockSpec((_BR, 1), lambda i: (i, 0)),
            pl.BlockSpec((1, HID), lambda i: (0, 0)),
            pl.BlockSpec((HID, HID), lambda i: (0, 0)),
        ],
        out_specs=pl.BlockSpec((2, _BR, HHID), lambda i: (0, i, 0)),
        out_shape=jax.ShapeDtypeStruct((2, N, HHID), jnp.float32),
    )(s1, s1, h1, h1, dis2, b1, W2)


def _comb2_body(sa_ref, sb_ref, ha_ref, hb_ref, d2_ref, b_ref, out_ref):
    d2 = d2_ref[...]
    b = b_ref[...]
    ta = sa_ref[0] + ha_ref[0] * d2 + b[:, :HHID]
    tb = sb_ref[0] + hb_ref[0] * d2 + b[:, HHID:]
    out_ref[...] = jnp.concatenate([ta, tb], axis=1)


def _comb2_call(s2, h2, dis2, b2):
    return pl.pallas_call(
        _comb2_body,
        grid=(_GRID,),
        in_specs=[
            pl.BlockSpec((1, _BR, HHID), lambda i: (0, i, 0)),
            pl.BlockSpec((1, _BR, HHID), lambda i: (1, i, 0)),
            pl.BlockSpec((1, _BR, HHID), lambda i: (0, i, 0)),
            pl.BlockSpec((1, _BR, HHID), lambda i: (1, i, 0)),
            pl.BlockSpec((_BR, 1), lambda i: (i, 0)),
            pl.BlockSpec((1, HID), lambda i: (0, 0)),
        ],
        out_specs=pl.BlockSpec((_BR, HID), lambda i: (i, 0)),
        out_shape=jax.ShapeDtypeStruct((N, HID), jnp.float32),
    )(s2, s2, h2, h2, dis2, b2)


def kernel(x, level, edge_index, edge_weight, W1, b1, W2, b2):
    del level
    src = edge_index[0]
    dst = edge_index[1]
    src_r = src.reshape(NS, NBLK, BLK_CH, CHUNK)
    dst_r = dst.reshape(NS, NBLK, BLK_CH, CHUNK)
    src_c = src.reshape(NS, NC, E_T32)
    dst_c = dst.reshape(NS, NC, E_T32)
    ew_c = edge_weight.reshape(NS, NC, E_T32)

    coef, dis2g = _coef_call(src_c, dst_c, ew_c)
    coef_r = coef.reshape(NS, NBLK, BLK_CH, CHUNK)
    dis2 = dis2g.reshape(NROW * 16)[:N, None]
    b1r = b1.reshape(1, HID)
    b2r = b2.reshape(1, HID)

    h1 = _mm_call(x, W1)
    s1 = _spmm_call(h1, src_r, dst_r, coef_r)
    h2 = _comb1_call(s1, h1, dis2, b1r, W2)
    o = _spmm_comb_call(h2, src_r, dst_r, coef_r, dis2g, b2.reshape(NC, HHID))
    return jnp.concatenate([o[0, :N], o[1, :N]], axis=1)


# speedup vs baseline: 1.0372x; 1.0372x over previous
import jax
import jax.numpy as jnp
from jax import lax
from jax.experimental import pallas as pl
from jax.experimental.pallas import tpu as pltpu
from jax.experimental.pallas import tpu_sc as plsc

N = 10000
E = 320000
FEAT = 128
HID = 128
HHID = HID // 2

NC = 2
NS = 16
E_T = E // NS
CHUNK = 80
BLK_CH = 25
NBLK = E_T // (BLK_CH * CHUNK)
N_PAD = 10240
ROWS_T = N_PAD // NS
TAB_STRIDE = 624

_mesh = plsc.VectorSubcoreMesh(core_axis_name="c", subcore_axis_name="s")
_sc_params = pltpu.CompilerParams(needs_layout_passes=False,
                                  use_tc_tiling_on_sc=False)


E_T32 = E // (NC * NS)
E_T16 = E // NS
NROW = 640
RPT = NROW // NS


def _frsqrt(x):
    i = lax.bitcast_convert_type(x, jnp.int32)
    i = jnp.int32(0x5F3759DF) - (i >> 1)
    y = lax.bitcast_convert_type(i, jnp.float32)
    for _ in range(3):
        y = y * (1.5 - 0.5 * x * y * y)
    return y


def _coef_body(src_hbm, dst_hbm, ew_hbm, coef_hbm, dis2_hbm,
               dst_v, ew_v, src_v, hist, dis_l, dstr, ystr, d2str,
               cf_v, idx_v, zbuf, dacc, sdis, sem):
    c = lax.axis_index("c")
    s = lax.axis_index("s")

    pltpu.sync_copy(dst_hbm.at[s], dst_v)
    pltpu.sync_copy(ew_hbm.at[s], ew_v)
    pltpu.sync_copy(src_hbm.at[s, c], src_v)

    zero16 = jnp.zeros((16,), jnp.float32)

    def zhist(r, _):
        hist[r] = zero16
        return 0

    lax.fori_loop(0, NROW, zhist, 0)

    def zrow(r, _):
        zbuf[r] = zero16
        return 0

    lax.fori_loop(0, RPT, zrow, 0)

    def fidx(g, _):
        idx_v[pl.ds(g * 16, 16)] = lax.iota(jnp.int32, 16) + g * 16
        return 0

    lax.fori_loop(0, NROW // 16, fidx, 0)

    base = s * RPT
    pltpu.sync_copy(zbuf, dacc.at[pl.ds(base, RPT)])
    plsc.subcore_barrier()

    def accum(h):
        def grp(i, _):
            d16 = dst_v[h, pl.ds(i * 16, 16)]
            v16 = ew_v[h, pl.ds(i * 16, 16)]
            plsc.addupdate_scatter(hist, [d16 >> 4, d16 & 15], v16)
            return 0

        lax.fori_loop(0, E_T32 // 16, grp, 0)

    accum(0)
    accum(1)

    for k in range(5):
        pltpu.async_copy(hist.at[pl.ds(k * 128, 128)],
                         dacc.at[idx_v.at[pl.ds(k * 128, 128)]], sem,
                         add=True)
    for k in range(5):
        pltpu.make_async_copy(hist.at[pl.ds(0, 128)],
                              dacc.at[idx_v.at[pl.ds(0, 128)]], sem).wait()
    plsc.subcore_barrier()

    pltpu.sync_copy(dacc.at[pl.ds(base, RPT)], dstr)

    def dgrp(r, _):
        deg = dstr[r] + 1.0
        y = _frsqrt(deg)
        ystr[r] = y
        d2str[r] = y * y
        return 0

    lax.fori_loop(0, RPT, dgrp, 0)
    pltpu.sync_copy(ystr, sdis.at[pl.ds(base, RPT)])

    @pl.when(c == 0)
    def _():
        pltpu.sync_copy(d2str, dis2_hbm.at[pl.ds(base, RPT)])

    plsc.subcore_barrier()

    pltpu.sync_copy(sdis, dis_l)

    def cgrp(i, _):
        sl = pl.ds(i * 16, 16)
        s16 = src_v[sl]
        d16 = dst_v[c, sl]
        a = plsc.load_gather(dis_l, [s16 >> 4, s16 & 15])
        b = plsc.load_gather(dis_l, [d16 >> 4, d16 & 15])
        cf_v[sl] = ew_v[c, sl] * a * b
        return 0

    lax.fori_loop(0, E_T32 // 16, cgrp, 0)
    pltpu.sync_copy(cf_v, coef_hbm.at[s, c])


_coef_call = pl.kernel(
    _coef_body,
    out_type=(jax.ShapeDtypeStruct((NS, NC, E_T32), jnp.float32),
              jax.ShapeDtypeStruct((NROW, 16), jnp.float32)),
    mesh=_mesh,
    scratch_types=[
        pltpu.VMEM((NC, E_T32), jnp.int32),
        pltpu.VMEM((NC, E_T32), jnp.float32),
        pltpu.VMEM((E_T32,), jnp.int32),
        pltpu.VMEM((NROW, 16), jnp.float32),
        pltpu.VMEM((NROW, 16), jnp.float32),
        pltpu.VMEM((RPT, 16), jnp.float32),
        pltpu.VMEM((RPT, 16), jnp.float32),
        pltpu.VMEM((RPT, 16), jnp.float32),
        pltpu.VMEM((E_T32,), jnp.float32),
        pltpu.VMEM((NROW,), jnp.int32),
        pltpu.VMEM((RPT, 16), jnp.float32),
        pltpu.VMEM_SHARED((NROW, 16), jnp.float32),
        pltpu.VMEM_SHARED((NROW, 16), jnp.float32),
        pltpu.SemaphoreType.DMA,
    ],
    compiler_params=_sc_params,
)


def _spmm_body(g_hbm, src_hbm, dst_hbm, ew_hbm, out_hbm,
               src_b, dst_b, ew_b, g0, g1, s0, s1, tab, acc,
               sg0, sg1, ss0, ss1, si0, si1):
    c = lax.axis_index("c")
    s = lax.axis_index("s")
    sis = (si0, si1)

    def issue_idx(b, p):
        pltpu.async_copy(src_hbm.at[s, b], src_b.at[p], sis[p])
        pltpu.async_copy(dst_hbm.at[s, b], dst_b.at[p], sis[p])
        pltpu.async_copy(ew_hbm.at[s, b], ew_b.at[p], sis[p])

    def wait_idx(b, p):
        pltpu.make_async_copy(src_hbm.at[s, b], src_b.at[p], sis[p]).wait()
        pltpu.make_async_copy(dst_hbm.at[s, b], dst_b.at[p], sis[p]).wait()
        pltpu.make_async_copy(ew_hbm.at[s, b], ew_b.at[p], sis[p]).wait()

    issue_idx(0, 0)

    tstart = s * TAB_STRIDE
    pltpu.sync_copy(g_hbm.at[c, pl.ds(tstart, 640)], tab.at[pl.ds(tstart, 640)])

    def zrow(i, _):
        for f in range(HHID // 16):
            s0[i, pl.ds(f * 16, 16)] = jnp.zeros((16,), jnp.float32)
        return 0

    lax.fori_loop(0, CHUNK, zrow, 0)
    base = s * ROWS_T
    for k in range(ROWS_T // CHUNK):
        pltpu.sync_copy(s0, acc.at[pl.ds(base + k * CHUNK, CHUNK)])
    plsc.subcore_barrier()

    def gather(p, j, buf, sem):
        pltpu.async_copy(tab.at[src_b.at[p, j]], buf, sem)

    def wait_gather(p, buf, sem):
        pltpu.make_async_copy(tab.at[src_b.at[p, 0]], buf, sem).wait()

    def scatter(p, j, buf, sem):
        pltpu.async_copy(buf, acc.at[dst_b.at[p, j]], sem, add=True)

    def wait_scatter(p, buf, sem):
        pltpu.make_async_copy(buf, acc.at[dst_b.at[p, 0]], sem).wait()

    def scale(p, j, gbuf, sbuf):
        def grp(g, _):
            ew16 = ew_b[p, j, pl.ds(g * 16, 16)]
            for l in range(16):
                wgt = jnp.full((16,), ew16[l])
                e = g * 16 + l
                for f in range(HHID // 16):
                    sl = pl.ds(f * 16, 16)
                    sbuf[e, sl] = gbuf[e, sl] * wgt
            return 0

        lax.fori_loop(0, CHUNK // 16, grp, 0)

    def process_block(p):
        gather(p, 0, g0, sg0)
        gather(p, 1, g1, sg1)

        wait_gather(p, g0, sg0)
        scale(p, 0, g0, s0)
        scatter(p, 0, s0, ss0)
        gather(p, 2, g0, sg0)
        wait_gather(p, g1, sg1)
        scale(p, 1, g1, s1)
        scatter(p, 1, s1, ss1)
        gather(p, 3, g1, sg1)

        def steady(i, _):
            j0 = 2 * i
            wait_gather(p, g0, sg0)
            wait_scatter(p, s0, ss0)
            scale(p, j0, g0, s0)
            scatter(p, j0, s0, ss0)
            gather(p, j0 + 2, g0, sg0)
            j1 = j0 + 1
            wait_gather(p, g1, sg1)
            wait_scatter(p, s1, ss1)
            scale(p, j1, g1, s1)
            scatter(p, j1, s1, ss1)
            gather(p, j1 + 2, g1, sg1)
            return 0

        lax.fori_loop(1, (BLK_CH - 3) // 2, steady, 0)

        wait_gather(p, g0, sg0)
        wait_scatter(p, s0, ss0)
        scale(p, BLK_CH - 3, g0, s0)
        scatter(p, BLK_CH - 3, s0, ss0)
        gather(p, BLK_CH - 1, g0, sg0)
        wait_gather(p, g1, sg1)
        wait_scatter(p, s1, ss1)
        scale(p, BLK_CH - 2, g1, s1)
        scatter(p, BLK_CH - 2, s1, ss1)
        wait_gather(p, g0, sg0)
        wait_scatter(p, s0, ss0)
        scale(p, BLK_CH - 1, g0, s0)
        scatter(p, BLK_CH - 1, s0, ss0)
        wait_scatter(p, s0, ss0)
        wait_scatter(p, s1, ss1)

    def pair_body(bb, _):
        b0 = 2 * bb
        b1 = b0 + 1
        bn = lax.rem(b0 + 2, NBLK)
        wait_idx(b0, 0)
        issue_idx(b1, 1)
        process_block(0)
        wait_idx(b1, 1)
        issue_idx(bn, 0)
        process_block(1)
        return 0

    lax.fori_loop(0, NBLK // 2, pair_body, 0)
    wait_idx(0, 0)

    plsc.subcore_barrier()
    pltpu.sync_copy(acc.at[pl.ds(base, ROWS_T)],
                    out_hbm.at[c, pl.ds(base, ROWS_T)])


_spmm_call = pl.kernel(
    _spmm_body,
    out_type=jax.ShapeDtypeStruct((NC, N_PAD, HHID), jnp.float32),
    mesh=_mesh,
    scratch_types=[
        pltpu.VMEM((2, BLK_CH, CHUNK), jnp.int32),
        pltpu.VMEM((2, BLK_CH, CHUNK), jnp.int32),
        pltpu.VMEM((2, BLK_CH, CHUNK), jnp.float32),
        pltpu.VMEM((CHUNK, HHID), jnp.float32),
        pltpu.VMEM((CHUNK, HHID), jnp.float32),
        pltpu.VMEM((CHUNK, HHID), jnp.float32),
        pltpu.VMEM((CHUNK, HHID), jnp.float32),
        pltpu.VMEM_SHARED((N_PAD, HHID), jnp.float32),
        pltpu.VMEM_SHARED((N_PAD, HHID), jnp.float32),
        pltpu.SemaphoreType.DMA,
        pltpu.SemaphoreType.DMA,
        pltpu.SemaphoreType.DMA,
        pltpu.SemaphoreType.DMA,
        pltpu.SemaphoreType.DMA,
        pltpu.SemaphoreType.DMA,
    ],
    compiler_params=_sc_params,
)


_BR = 1000
_GRID = N // _BR


def _halves(v):
    return v[:, :HHID], v[:, HHID:]


def _mm_body(x_ref, w_ref, h_ref):
    h = jnp.dot(x_ref[...], w_ref[...], preferred_element_type=jnp.float32,
                precision=lax.Precision.HIGHEST)
    ha, hb = _halves(h)
    h_ref[0] = ha
    h_ref[1] = hb


def _mm_call(x, W1):
    return pl.pallas_call(
        _mm_body,
        grid=(_GRID,),
        in_specs=[
            pl.BlockSpec((_BR, FEAT), lambda i: (i, 0)),
            pl.BlockSpec((FEAT, HID), lambda i: (0, 0)),
        ],
        out_specs=pl.BlockSpec((2, _BR, HHID), lambda i: (0, i, 0)),
        out_shape=jax.ShapeDtypeStruct((2, N, HHID), jnp.float32),
    )(x, W1)


def _comb1_body(sa_ref, sb_ref, ha_ref, hb_ref, d2_ref, b_ref, w_ref, h2_ref):
    d2 = d2_ref[...]
    b = b_ref[...]
    ta = sa_ref[0] + ha_ref[0] * d2 + b[:, :HHID]
    tb = sb_ref[0] + hb_ref[0] * d2 + b[:, HHID:]
    o = jnp.maximum(jnp.concatenate([ta, tb], axis=1), 0.0)
    h2 = jnp.dot(o, w_ref[...], preferred_element_type=jnp.float32,
                 precision=lax.Precision.HIGHEST)
    h2a, h2b = _halves(h2)
    h2_ref[0] = h2a
    h2_ref[1] = h2b


def _comb1_call(s1, h1, dis2, b1, W2):
    return pl.pallas_call(
        _comb1_body,
        grid=(_GRID,),
        in_specs=[
            pl.BlockSpec((1, _BR, HHID), lambda i: (0, i, 0)),
            pl.BlockSpec((1, _BR, HHID), lambda i: (1, i, 0)),
            pl.BlockSpec((1, _BR, HHID), lambda i: (0, i, 0)),
            pl.BlockSpec((1, _BR, HHID), lambda i: (1, i, 0)),
            pl.BlockSpec((_BR, 1), lambda i: (i, 0)),
            pl.BlockSpec((1, HID), lambda i: (0, 0)),
            pl.BlockSpec((HID, HID), lambda i: (0, 0)),
        ],
        out_specs=pl.BlockSpec((2, _BR, HHID), lambda i: (0, i, 0)),
        out_shape=jax.ShapeDtypeStruct((2, N, HHID), jnp.float32),
    )(s1, s1, h1, h1, dis2, b1, W2)


def _comb2_body(sa_ref, sb_ref, ha_ref, hb_ref, d2_ref, b_ref, out_ref):
    d2 = d2_ref[...]
    b = b_ref[...]
    ta = sa_ref[0] + ha_ref[0] * d2 + b[:, :HHID]
    tb = sb_ref[0] + hb_ref[0] * d2 + b[:, HHID:]
    out_ref[...] = jnp.concatenate([ta, tb], axis=1)


def _comb2_call(s2, h2, dis2, b2):
    return pl.pallas_call(
        _comb2_body,
        grid=(_GRID,),
        in_specs=[
            pl.BlockSpec((1, _BR, HHID), lambda i: (0, i, 0)),
            pl.BlockSpec((1, _BR, HHID), lambda i: (1, i, 0)),
            pl.BlockSpec((1, _BR, HHID), lambda i: (0, i, 0)),
            pl.BlockSpec((1, _BR, HHID), lambda i: (1, i, 0)),
            pl.BlockSpec((_BR, 1), lambda i: (i, 0)),
            pl.BlockSpec((1, HID), lambda i: (0, 0)),
        ],
        out_specs=pl.BlockSpec((_BR, HID), lambda i: (i, 0)),
        out_shape=jax.ShapeDtypeStruct((N, HID), jnp.float32),
    )(s2, s2, h2, h2, dis2, b2)


def kernel(x, level, edge_index, edge_weight, W1, b1, W2, b2):
    del level
    src = edge_index[0]
    dst = edge_index[1]
    src_r = src.reshape(NS, NBLK, BLK_CH, CHUNK)
    dst_r = dst.reshape(NS, NBLK, BLK_CH, CHUNK)
    src_c = src.reshape(NS, NC, E_T32)
    dst_c = dst.reshape(NS, NC, E_T32)
    ew_c = edge_weight.reshape(NS, NC, E_T32)

    coef, dis2g = _coef_call(src_c, dst_c, ew_c)
    coef_r = coef.reshape(NS, NBLK, BLK_CH, CHUNK)
    dis2 = dis2g.reshape(NROW * 16)[:N, None]
    b1r = b1.reshape(1, HID)
    b2r = b2.reshape(1, HID)

    h1 = _mm_call(x, W1)
    s1 = _spmm_call(h1, src_r, dst_r, coef_r)
    h2 = _comb1_call(s1, h1, dis2, b1r, W2)
    s2 = _spmm_call(h2, src_r, dst_r, coef_r)
    out = _comb2_call(s2, h2, dis2, b2r)
    return out
